# baseline (speedup 1.0000x reference)
import functools

import jax
import jax.numpy as jnp
from jax import lax
from jax.experimental import pallas as pl
from jax.experimental.pallas import tpu as pltpu
from jax.experimental.pallas import tpu_sc as plsc

DIM = 64
LANES = 16
NUM_CORES = 2
NUM_SUBCORES = 16
NUM_WORKERS = NUM_CORES * NUM_SUBCORES
CHUNK = 256


@functools.lru_cache(maxsize=None)
def _build(total, n_ent, n_rel):
    b_per_w = total // NUM_WORKERS
    n_chunks = b_per_w // CHUNK
    groups = CHUNK // LANES
    mesh = plsc.VectorSubcoreMesh(core_axis_name="c", subcore_axis_name="s")

    @functools.partial(
        pl.kernel,
        out_type=jax.ShapeDtypeStruct((total,), jnp.float32),
        mesh=mesh,
        compiler_params=pltpu.CompilerParams(needs_layout_passes=False,
                                             use_tc_tiling_on_sc=False),
        scratch_types=[
            pltpu.VMEM((b_per_w,), jnp.int32),
            pltpu.VMEM((b_per_w,), jnp.int32),
            pltpu.VMEM((b_per_w,), jnp.int32),
            pltpu.VMEM((CHUNK, DIM), jnp.float32),
            pltpu.VMEM((CHUNK, DIM), jnp.float32),
            pltpu.VMEM((CHUNK, DIM), jnp.float32),
            pltpu.VMEM((b_per_w,), jnp.float32),
            pltpu.SemaphoreType.DMA,
        ],
    )
    def score_kernel(h_hbm, r_hbm, t_hbm, ent_hbm, rel_hbm, out_hbm,
                     idx_h, idx_r, idx_t, hrow, rrow, trow, outv, sem):
        wid = lax.axis_index("s") * NUM_CORES + lax.axis_index("c")
        base = wid * b_per_w
        pltpu.sync_copy(h_hbm.at[pl.ds(base, b_per_w)], idx_h)
        pltpu.sync_copy(r_hbm.at[pl.ds(base, b_per_w)], idx_r)
        pltpu.sync_copy(t_hbm.at[pl.ds(base, b_per_w)], idx_t)
        iota = lax.iota(jnp.int32, LANES)

        for c in range(n_chunks):
            off = c * CHUNK
            cp1 = pltpu.async_copy(ent_hbm.at[idx_h.at[pl.ds(off, CHUNK)]], hrow, sem)
            cp2 = pltpu.async_copy(rel_hbm.at[idx_r.at[pl.ds(off, CHUNK)]], rrow, sem)
            cp3 = pltpu.async_copy(ent_hbm.at[idx_t.at[pl.ds(off, CHUNK)]], trow, sem)
            cp1.wait()
            cp2.wait()
            cp3.wait()

            def group_body(g, _, off=off):
                rows = g * LANES + iota
                accs = [jnp.zeros((LANES,), jnp.float32) for _ in range(4)]
                for d in range(DIM):
                    dd = jnp.full((LANES,), d, jnp.int32)
                    hv = plsc.load_gather(hrow, [rows, dd])
                    rv = plsc.load_gather(rrow, [rows, dd])
                    tv = plsc.load_gather(trow, [rows, dd])
                    accs[d % 4] = accs[d % 4] + hv * rv * tv
                outv[pl.ds(off + g * LANES, LANES)] = (
                    (accs[0] + accs[1]) + (accs[2] + accs[3]))
                return _

            lax.fori_loop(0, groups, group_body, None)

        pltpu.sync_copy(outv, out_hbm.at[pl.ds(base, b_per_w)])

    return score_kernel


def kernel(pos_triples, neg_triples, entity_weight, relation_weight):
    batch = pos_triples.shape[0]
    trip = jnp.concatenate([pos_triples, neg_triples], axis=0)
    h = trip[:, 0]
    r = trip[:, 1]
    t = trip[:, 2]
    scores = _build(2 * batch, entity_weight.shape[0], relation_weight.shape[0])(
        h, r, t, entity_weight, relation_weight)
    return scores[:batch], scores[batch:]
